# Initial kernel scaffold; baseline (speedup 1.0000x reference)
import jax
import jax.numpy as jnp
from jax.experimental import pallas as pl


def kernel(x, pos_emb):
    raise NotImplementedError("write your pallas kernel here")


# speedup vs baseline: 1.4929x; 1.4929x over previous
import jax
import jax.numpy as jnp
from jax.experimental import pallas as pl


_TT = 512


def _add_body(x_ref, pe_ref, o_ref):
    o_ref[0] = x_ref[0] + pe_ref[...]


def kernel(x, pos_emb):
    B, T, C = x.shape
    grid = (T // _TT, B)
    return pl.pallas_call(
        _add_body,
        grid=grid,
        in_specs=[
            pl.BlockSpec((1, _TT, C), lambda i, j: (j, i, 0)),
            pl.BlockSpec((_TT, C), lambda i, j: (i, 0)),
        ],
        out_specs=pl.BlockSpec((1, _TT, C), lambda i, j: (j, i, 0)),
        out_shape=jax.ShapeDtypeStruct((B, T, C), x.dtype),
    )(x, pos_emb)
